# Initial kernel scaffold; baseline (speedup 1.0000x reference)
import jax
import jax.numpy as jnp
from jax.experimental import pallas as pl


def kernel(x, wg, wn, Wgate, Wup, Wdown):
    raise NotImplementedError("write your pallas kernel here")


# speedup vs baseline: 1.4738x; 1.4738x over previous
import jax
import jax.numpy as jnp
from jax.experimental import pallas as pl
from jax.experimental.pallas import tpu as pltpu

E = 8
K = 2
D = 768
F = 3072
T = 2048
FC = 768
NC = F // FC


def _router_body(x_ref, wg_ref, wn_ref, eps_ref, gates_ref):
    x = x_ref[...]
    dn = (((1,), (1,)), ((), ()))
    logits = jax.lax.dot_general(x, wg_ref[...], dn,
                                 preferred_element_type=jnp.float32)
    zn = jax.lax.dot_general(x, wn_ref[...], dn,
                             preferred_element_type=jnp.float32)
    sp = jnp.maximum(zn, 0.0) + jnp.log(1.0 + jnp.exp(-jnp.abs(zn)))
    logits = logits + eps_ref[...] * sp
    idx = jax.lax.broadcasted_iota(jnp.int32, logits.shape, 1)
    m1 = jnp.max(logits, axis=1, keepdims=True)
    i1 = jnp.min(jnp.where(logits == m1, idx, E), axis=1, keepdims=True)
    masked = jnp.where(idx == i1, -jnp.inf, logits)
    m2 = jnp.max(masked, axis=1, keepdims=True)
    i2 = jnp.min(jnp.where(masked == m2, idx, E), axis=1, keepdims=True)
    e2v = jnp.exp(m2 - m1)
    g1 = 1.0 / (1.0 + e2v)
    g2 = e2v * g1
    gates = jnp.where(idx == i1, g1, jnp.where(idx == i2, g2, 0.0))
    gates_ref[...] = gates


def _router(x, wg, wn):
    eps = jax.random.normal(jax.random.key(42), (T, E), dtype=jnp.float32)
    return pl.pallas_call(
        _router_body,
        out_shape=jax.ShapeDtypeStruct((T, E), jnp.float32),
    )(x, wg, wn, eps)


def _ffn_body(gates_ref, x_ref, wg_ref, wu_ref, wd_ref, out_ref):
    c = pl.program_id(1)
    first = (pl.program_id(0) == 0) & (c == 0)
    dn = (((1,), (1,)), ((), ()))
    x = x_ref[...]
    g = jax.lax.dot_general(x, wg_ref[0], dn,
                            preferred_element_type=jnp.float32)
    g = g * (1.0 / (1.0 + jnp.exp(-g)))
    u = jax.lax.dot_general(x, wu_ref[0], dn,
                            preferred_element_type=jnp.float32)
    h = g * u
    o = jax.lax.dot_general(h, wd_ref[0], dn,
                            preferred_element_type=jnp.float32)
    o = o * gates_ref[0, 0][:, None]

    @pl.when(first)
    def _():
        out_ref[...] = o

    @pl.when(jnp.logical_not(first))
    def _():
        out_ref[...] += o


def _ffn(gates_t, x, Wgate, Wup, Wdown):
    return pl.pallas_call(
        _ffn_body,
        grid=(E, NC),
        in_specs=[
            pl.BlockSpec((1, 1, T), lambda e, c: (e, 0, 0)),
            pl.BlockSpec((T, D), lambda e, c: (0, 0)),
            pl.BlockSpec((1, FC, D), lambda e, c: (e, c, 0)),
            pl.BlockSpec((1, FC, D), lambda e, c: (e, c, 0)),
            pl.BlockSpec((1, D, FC), lambda e, c: (e, 0, c)),
        ],
        out_specs=pl.BlockSpec((T, D), lambda e, c: (0, 0)),
        out_shape=jax.ShapeDtypeStruct((T, D), jnp.float32),
        compiler_params=pltpu.CompilerParams(
            dimension_semantics=("arbitrary", "arbitrary"),
        ),
    )(gates_t, x, Wgate, Wup, Wdown)


def kernel(x, wg, wn, Wgate, Wup, Wdown):
    gates = _router(x, wg, wn)
    gates_t = gates.T.reshape(E, 1, T)
    return _ffn(gates_t, x, Wgate, Wup, Wdown)
